# Initial kernel scaffold; baseline (speedup 1.0000x reference)
import jax
import jax.numpy as jnp
from jax.experimental import pallas as pl


def kernel(x, edge_index, edge_attr, batch, embed, W1, b1, W2, b2, W3, b3, lin_W, lin_b):
    raise NotImplementedError("write your pallas kernel here")


# speedup vs baseline: 6.6224x; 6.6224x over previous
import functools

import jax
import jax.numpy as jnp
from jax import lax
from jax.experimental import pallas as pl
from jax.experimental.pallas import tpu as pltpu
from jax.experimental.pallas import tpu_sc as plsc

_N = 100000
_NPAD = 102400
_HID = 32
_OUT = 16
_E2D = 12544
_EPAD = _E2D * 128
_HALF = 50000
_DUMP = 50000
_ACC_ROWS = 51200
_RPW = _E2D // 16
_NCHUNK = _RPW // 8
_PPW = 25
_PROWS = 72
_PCOLS = 48
_HHID = 16

_mesh = plsc.VectorSubcoreMesh(core_axis_name="c", subcore_axis_name="s")


@functools.partial(
    pl.kernel,
    out_type=[
        jax.ShapeDtypeStruct((_NPAD, _HID), jnp.float32),
        jax.ShapeDtypeStruct((_NPAD, _HHID), jnp.float32),
        jax.ShapeDtypeStruct((_E2D, 128), jnp.int32),
        jax.ShapeDtypeStruct((_E2D, 128), jnp.int32),
    ],
    mesh=_mesh,
    compiler_params=pltpu.CompilerParams(use_tc_tiling_on_sc=False, needs_layout_passes=False),
    scratch_types=[
        pltpu.VMEM((_PPW * 128,), jnp.int32),
        pltpu.VMEM((640, _HID), jnp.float32),
        pltpu.VMEM((8, 128), jnp.int32),
        pltpu.VMEM((8, 128), jnp.int32),
        pltpu.VMEM((128, _HHID), jnp.float32),
        pltpu.VMEM((640, _HHID), jnp.float32),
        pltpu.VMEM_SHARED((_ACC_ROWS, _HHID), jnp.float32),
        pltpu.SemaphoreType.DMA,
    ],
)
def _sc_embed_deg(embed, x1d, dst2d, ones1, zeros1,
                  h0, deg, lidx0, lidx1,
                  idxv, rows, dstb, lidxb, onesv, vbuf, accdeg, sem):
    c = lax.axis_index("c")
    s = lax.axis_index("s")
    w = s * 2 + c

    pltpu.sync_copy(x1d.at[pl.ds(w * 3200, 3200)], idxv)

    def eround(r, carry):
        cps = [
            pltpu.async_copy(embed.at[idxv.at[pl.ds(r * 640 + j * 128, 128)]],
                             rows.at[pl.ds(j * 128, 128)], sem)
            for j in range(5)
        ]
        for cp in cps:
            cp.wait()
        pltpu.sync_copy(rows, h0.at[pl.ds(w * 3200 + r * 640, 640)])
        return carry

    lax.fori_loop(0, 5, eround, 0)

    pltpu.sync_copy(ones1, onesv)
    pltpu.sync_copy(zeros1, vbuf)
    for kk in range(5):
        pltpu.sync_copy(vbuf, accdeg.at[pl.ds(s * 3200 + kk * 640, 640)])
    plsc.subcore_barrier()
    base = c * _HALF

    def chunk(i, carry):
        row0 = s * _RPW + i * 8
        pltpu.sync_copy(dst2d.at[pl.ds(row0, 8)], dstb)
        for j in range(8):
            for k in range(8):
                d = dstb[j, pl.ds(k * 16, 16)]
                m = (d >= base) & (d < base + _HALF)
                lidxb[j, pl.ds(k * 16, 16)] = jnp.where(m, d - base, _DUMP)

        @pl.when(c == 0)
        def _():
            pltpu.sync_copy(lidxb, lidx0.at[pl.ds(row0, 8)])

        @pl.when(c == 1)
        def _():
            pltpu.sync_copy(lidxb, lidx1.at[pl.ds(row0, 8)])

        for j in range(8):
            pltpu.sync_copy(onesv, accdeg.at[lidxb.at[j]], add=True)
        return carry

    lax.fori_loop(0, _NCHUNK, chunk, 0)
    plsc.subcore_barrier()

    @pl.when(s < 15)
    def _():
        for kk in range(5):
            pltpu.sync_copy(accdeg.at[pl.ds(s * 3200 + kk * 640, 640)], vbuf)
            pltpu.sync_copy(vbuf, deg.at[pl.ds(c * _HALF + s * 3200 + kk * 640, 640)])

    @pl.when(s == 15)
    def _():
        for kk in range(5):
            pltpu.sync_copy(accdeg.at[pl.ds(48000 + kk * 400, 400)],
                            vbuf.at[pl.ds(0, 400)])
            pltpu.sync_copy(vbuf.at[pl.ds(0, 400)],
                            deg.at[pl.ds(c * _HALF + 48000 + kk * 400, 400)])


@functools.partial(
    pl.kernel,
    out_type=jax.ShapeDtypeStruct((_NPAD, _HHID), jnp.float32),
    mesh=_mesh,
    compiler_params=pltpu.CompilerParams(use_tc_tiling_on_sc=False, needs_layout_passes=False),
    scratch_types=[
        pltpu.VMEM((8, 128), jnp.int32),
        pltpu.VMEM((8, 128), jnp.int32),
        pltpu.VMEM((1024, _HHID), jnp.float32),
        pltpu.VMEM((640, _HHID), jnp.float32),
        pltpu.VMEM_SHARED((_ACC_ROWS, _HHID), jnp.float32),
        pltpu.SemaphoreType.DMA,
    ],
)
def _sc_scatter(p, src2d, lidx0, lidx1, zerosf, acc_out,
                srcb, lidxb, rows, vbuf, accf, sem):
    c = lax.axis_index("c")
    s = lax.axis_index("s")
    pltpu.sync_copy(zerosf, vbuf)
    for kk in range(5):
        pltpu.sync_copy(vbuf, accf.at[pl.ds(s * 3200 + kk * 640, 640)])
    plsc.subcore_barrier()

    def chunk(i, carry):
        row0 = s * _RPW + i * 8
        pltpu.sync_copy(src2d.at[pl.ds(row0, 8)], srcb)

        @pl.when(c == 0)
        def _():
            pltpu.sync_copy(lidx0.at[pl.ds(row0, 8)], lidxb)

        @pl.when(c == 1)
        def _():
            pltpu.sync_copy(lidx1.at[pl.ds(row0, 8)], lidxb)

        cps = [
            pltpu.async_copy(p.at[srcb.at[j]], rows.at[pl.ds(j * 128, 128)], sem)
            for j in range(8)
        ]
        for cp in cps:
            cp.wait()
        for j in range(8):
            pltpu.sync_copy(rows.at[pl.ds(j * 128, 128)],
                            accf.at[lidxb.at[j]], add=True)
        return carry

    lax.fori_loop(0, _NCHUNK, chunk, 0)
    plsc.subcore_barrier()

    @pl.when(s < 15)
    def _():
        for kk in range(5):
            pltpu.sync_copy(accf.at[pl.ds(s * 3200 + kk * 640, 640)], vbuf)
            pltpu.sync_copy(vbuf, acc_out.at[pl.ds(c * _HALF + s * 3200 + kk * 640, 640)])

    @pl.when(s == 15)
    def _():
        for kk in range(5):
            pltpu.sync_copy(accf.at[pl.ds(48000 + kk * 400, 400)],
                            vbuf.at[pl.ds(0, 400)])
            pltpu.sync_copy(vbuf.at[pl.ds(0, 400)],
                            acc_out.at[pl.ds(c * _HALF + 48000 + kk * 400, 400)])


@functools.partial(
    pl.kernel,
    out_type=jax.ShapeDtypeStruct((32 * _PROWS * _PCOLS,), jnp.float32),
    mesh=_mesh,
    compiler_params=pltpu.CompilerParams(use_tc_tiling_on_sc=False, needs_layout_passes=False),
    scratch_types=[
        pltpu.VMEM((128, _HID), jnp.float32),
        pltpu.VMEM((128,), jnp.int32),
        pltpu.VMEM((_PROWS * _PCOLS,), jnp.float32),
    ],
)
def _sc_pool(h3, batch1d, zerosp, part, hbuf, bbuf, accp):
    c = lax.axis_index("c")
    s = lax.axis_index("s")
    w = s * 2 + c
    pltpu.sync_copy(zerosp, accp)
    lanes = lax.iota(jnp.int32, 16)
    cntv = jnp.where(lanes == 0, 1.0, 0.0)

    def chunk(i, carry):
        pltpu.sync_copy(h3.at[pl.ds(w * 3200 + i * 128, 128)], hbuf)
        pltpu.sync_copy(batch1d.at[pl.ds(w * 3200 + i * 128, 128)], bbuf)
        for k in range(8):
            bvec = bbuf[pl.ds(k * 16, 16)]
            for j in range(16):
                n = k * 16 + j
                b_s = jnp.sum(jnp.where(lanes == j, bvec, 0))
                idx0 = b_s * _PCOLS + lanes
                plsc.addupdate_scatter(accp, [idx0], hbuf[n, pl.ds(0, 16)])
                plsc.addupdate_scatter(accp, [idx0 + 16], hbuf[n, pl.ds(16, 16)])
                plsc.addupdate_scatter(accp, [idx0 + 32], cntv)
        return carry

    lax.fori_loop(0, _PPW, chunk, 0)
    pltpu.sync_copy(accp, part.at[pl.ds(w * (_PROWS * _PCOLS), _PROWS * _PCOLS)])


_BLK = 3200


def _tc0_body(deg_ref, h_ref, w_ref, pl_ref, ph_ref, dinv_ref):
    i = pl.program_id(0)
    gid = i * _BLK + jax.lax.broadcasted_iota(jnp.int32, (_BLK, 1), 0)
    real = gid < _N
    dv = jnp.where(real, lax.rsqrt(deg_ref[:, :1] + 1.0), 0.0)
    p = jnp.dot(h_ref[...], w_ref[...],
                preferred_element_type=jnp.float32) * dv
    pl_ref[...] = p[:, :_HHID]
    ph_ref[...] = p[:, _HHID:]
    dinv_ref[...] = dv


def _tc0(h0, deg, W1):
    return pl.pallas_call(
        _tc0_body,
        grid=(_NPAD // _BLK,),
        in_specs=[
            pl.BlockSpec((_BLK, _HHID), lambda i: (i, 0)),
            pl.BlockSpec((_BLK, _HID), lambda i: (i, 0)),
            pl.BlockSpec((_HID, _HID), lambda i: (0, 0)),
        ],
        out_specs=[
            pl.BlockSpec((_BLK, _HHID), lambda i: (i, 0)),
            pl.BlockSpec((_BLK, _HHID), lambda i: (i, 0)),
            pl.BlockSpec((_BLK, 1), lambda i: (i, 0)),
        ],
        out_shape=[
            jax.ShapeDtypeStruct((_NPAD, _HHID), jnp.float32),
            jax.ShapeDtypeStruct((_NPAD, _HHID), jnp.float32),
            jax.ShapeDtypeStruct((_NPAD, 1), jnp.float32),
        ],
    )(deg, h0, W1)


def _tc_mid_body(al_ref, ah_ref, pl_ref, ph_ref, dinv_ref, b_ref, w_ref,
                 ol_ref, oh_ref):
    dv = dinv_ref[...]
    acc = jnp.concatenate([al_ref[...], ah_ref[...]], axis=1)
    p = jnp.concatenate([pl_ref[...], ph_ref[...]], axis=1)
    h = jnp.maximum(dv * (acc + p) + b_ref[...], 0.0)
    pn = jnp.dot(h, w_ref[...], preferred_element_type=jnp.float32) * dv
    ol_ref[...] = pn[:, :_HHID]
    oh_ref[...] = pn[:, _HHID:]


def _tc_mid(al, ah, p_lo, p_hi, dinv, b, W):
    return pl.pallas_call(
        _tc_mid_body,
        grid=(_NPAD // _BLK,),
        in_specs=[
            pl.BlockSpec((_BLK, _HHID), lambda i: (i, 0)),
            pl.BlockSpec((_BLK, _HHID), lambda i: (i, 0)),
            pl.BlockSpec((_BLK, _HHID), lambda i: (i, 0)),
            pl.BlockSpec((_BLK, _HHID), lambda i: (i, 0)),
            pl.BlockSpec((_BLK, 1), lambda i: (i, 0)),
            pl.BlockSpec((1, _HID), lambda i: (0, 0)),
            pl.BlockSpec((_HID, _HID), lambda i: (0, 0)),
        ],
        out_specs=[
            pl.BlockSpec((_BLK, _HHID), lambda i: (i, 0)),
            pl.BlockSpec((_BLK, _HHID), lambda i: (i, 0)),
        ],
        out_shape=[
            jax.ShapeDtypeStruct((_NPAD, _HHID), jnp.float32),
            jax.ShapeDtypeStruct((_NPAD, _HHID), jnp.float32),
        ],
    )(al, ah, p_lo, p_hi, dinv, b, W)


def _tc_h3_body(al_ref, ah_ref, pl_ref, ph_ref, dinv_ref, b_ref, out_ref):
    acc = jnp.concatenate([al_ref[...], ah_ref[...]], axis=1)
    p = jnp.concatenate([pl_ref[...], ph_ref[...]], axis=1)
    out_ref[...] = dinv_ref[...] * (acc + p) + b_ref[...]


def _tc_h3(al, ah, p_lo, p_hi, dinv, b):
    return pl.pallas_call(
        _tc_h3_body,
        grid=(_NPAD // _BLK,),
        in_specs=[
            pl.BlockSpec((_BLK, _HHID), lambda i: (i, 0)),
            pl.BlockSpec((_BLK, _HHID), lambda i: (i, 0)),
            pl.BlockSpec((_BLK, _HHID), lambda i: (i, 0)),
            pl.BlockSpec((_BLK, _HHID), lambda i: (i, 0)),
            pl.BlockSpec((_BLK, 1), lambda i: (i, 0)),
            pl.BlockSpec((1, _HID), lambda i: (0, 0)),
        ],
        out_specs=pl.BlockSpec((_BLK, _HID), lambda i: (i, 0)),
        out_shape=jax.ShapeDtypeStruct((_NPAD, _HID), jnp.float32),
    )(al, ah, p_lo, p_hi, dinv, b)


def _tc4_body(part_ref, lw_ref, lb_ref, out_ref):
    t = jnp.sum(part_ref[...], axis=0)
    sums = t[:64, :_HID]
    cnt = t[:64, _HID:_HID + 1]
    pooled = sums / jnp.maximum(cnt, 1.0)
    out_ref[...] = jnp.dot(pooled, lw_ref[...],
                           preferred_element_type=jnp.float32) + lb_ref[...]


def _tc4(part, lin_W, lin_b):
    return pl.pallas_call(
        _tc4_body,
        out_shape=jax.ShapeDtypeStruct((64, _OUT), jnp.float32),
    )(part, lin_W, lin_b)


def kernel(x, edge_index, edge_attr, batch, embed,
           W1, b1, W2, b2, W3, b3, lin_W, lin_b):
    del edge_attr
    xp = jnp.concatenate([x[:, 0], jnp.zeros((_NPAD - _N,), jnp.int32)])
    src = edge_index[0]
    dst = edge_index[1]
    pad_e = _EPAD - src.shape[0]
    src2d = jnp.concatenate(
        [src, jnp.zeros((pad_e,), jnp.int32)]).reshape(_E2D, 128)
    dst2d = jnp.concatenate(
        [dst, jnp.full((pad_e,), 1 << 20, jnp.int32)]).reshape(_E2D, 128)
    batchp = jnp.concatenate([batch, jnp.full((_NPAD - _N,), 64, jnp.int32)])
    ones1 = jnp.ones((128, _HHID), jnp.float32)
    zeros1 = jnp.zeros((640, _HHID), jnp.float32)
    zerosf = jnp.zeros((640, _HHID), jnp.float32)
    zerosp = jnp.zeros((_PROWS * _PCOLS,), jnp.float32)

    h0, deg, lidx0, lidx1 = _sc_embed_deg(embed, xp, dst2d, ones1, zeros1)
    p1l, p1h, dinv = _tc0(h0, deg, W1)
    a1l = _sc_scatter(p1l, src2d, lidx0, lidx1, zerosf)
    a1h = _sc_scatter(p1h, src2d, lidx0, lidx1, zerosf)
    p2l, p2h = _tc_mid(a1l, a1h, p1l, p1h, dinv, b1.reshape(1, _HID), W2)
    a2l = _sc_scatter(p2l, src2d, lidx0, lidx1, zerosf)
    a2h = _sc_scatter(p2h, src2d, lidx0, lidx1, zerosf)
    p3l, p3h = _tc_mid(a2l, a2h, p2l, p2h, dinv, b2.reshape(1, _HID), W3)
    a3l = _sc_scatter(p3l, src2d, lidx0, lidx1, zerosf)
    a3h = _sc_scatter(p3h, src2d, lidx0, lidx1, zerosf)
    h3 = _tc_h3(a3l, a3h, p3l, p3h, dinv, b3.reshape(1, _HID))
    part = _sc_pool(h3, batchp, zerosp)
    out = _tc4(part.reshape(32, _PROWS, _PCOLS), lin_W, lin_b.reshape(1, _OUT))
    return out


# speedup vs baseline: 6.7105x; 1.0133x over previous
import functools

import jax
import jax.numpy as jnp
from jax import lax
from jax.experimental import pallas as pl
from jax.experimental.pallas import tpu as pltpu
from jax.experimental.pallas import tpu_sc as plsc

_N = 100000
_NPAD = 102400
_HID = 32
_OUT = 16
_E2D = 12544
_EPAD = _E2D * 128
_HALF = 50000
_DUMP = 50000
_ACC_ROWS = 51200
_RPW = _E2D // 16
_NCHUNK = _RPW // 8
_PPW = 25
_PROWS = 72
_PCOLS = 48
_HHID = 16

_mesh = plsc.VectorSubcoreMesh(core_axis_name="c", subcore_axis_name="s")


@functools.partial(
    pl.kernel,
    out_type=[
        jax.ShapeDtypeStruct((_NPAD, _HID), jnp.float32),
        jax.ShapeDtypeStruct((_NPAD, _HHID), jnp.float32),
        jax.ShapeDtypeStruct((_E2D, 128), jnp.int32),
        jax.ShapeDtypeStruct((_E2D, 128), jnp.int32),
    ],
    mesh=_mesh,
    compiler_params=pltpu.CompilerParams(use_tc_tiling_on_sc=False, needs_layout_passes=False),
    scratch_types=[
        pltpu.VMEM((_PPW * 128,), jnp.int32),
        pltpu.VMEM((640, _HID), jnp.float32),
        pltpu.VMEM((8, 128), jnp.int32),
        pltpu.VMEM((8, 128), jnp.int32),
        pltpu.VMEM((128, _HHID), jnp.float32),
        pltpu.VMEM((640, _HHID), jnp.float32),
        pltpu.VMEM_SHARED((_ACC_ROWS, _HHID), jnp.float32),
        pltpu.SemaphoreType.DMA,
    ],
)
def _sc_embed_deg(embed, x1d, dst2d, ones1, zeros1,
                  h0, deg, lidx0, lidx1,
                  idxv, rows, dstb, lidxb, onesv, vbuf, accdeg, sem):
    c = lax.axis_index("c")
    s = lax.axis_index("s")
    w = s * 2 + c

    pltpu.sync_copy(x1d.at[pl.ds(w * 3200, 3200)], idxv)

    def eround(r, carry):
        cps = [
            pltpu.async_copy(embed.at[idxv.at[pl.ds(r * 640 + j * 128, 128)]],
                             rows.at[pl.ds(j * 128, 128)], sem)
            for j in range(5)
        ]
        for cp in cps:
            cp.wait()
        pltpu.sync_copy(rows, h0.at[pl.ds(w * 3200 + r * 640, 640)])
        return carry

    lax.fori_loop(0, 5, eround, 0)

    pltpu.sync_copy(ones1, onesv)
    pltpu.sync_copy(zeros1, vbuf)
    for kk in range(5):
        pltpu.sync_copy(vbuf, accdeg.at[pl.ds(s * 3200 + kk * 640, 640)])
    plsc.subcore_barrier()
    base = c * _HALF

    def chunk(i, carry):
        row0 = s * _RPW + i * 8
        pltpu.sync_copy(dst2d.at[pl.ds(row0, 8)], dstb)
        for j in range(8):
            for k in range(8):
                d = dstb[j, pl.ds(k * 16, 16)]
                m = (d >= base) & (d < base + _HALF)
                lidxb[j, pl.ds(k * 16, 16)] = jnp.where(m, d - base, _DUMP)

        @pl.when(c == 0)
        def _():
            pltpu.sync_copy(lidxb, lidx0.at[pl.ds(row0, 8)])

        @pl.when(c == 1)
        def _():
            pltpu.sync_copy(lidxb, lidx1.at[pl.ds(row0, 8)])

        for j in range(8):
            pltpu.sync_copy(onesv, accdeg.at[lidxb.at[j]], add=True)
        return carry

    lax.fori_loop(0, _NCHUNK, chunk, 0)
    plsc.subcore_barrier()

    @pl.when(s < 15)
    def _():
        for kk in range(5):
            pltpu.sync_copy(accdeg.at[pl.ds(s * 3200 + kk * 640, 640)], vbuf)
            pltpu.sync_copy(vbuf, deg.at[pl.ds(c * _HALF + s * 3200 + kk * 640, 640)])

    @pl.when(s == 15)
    def _():
        for kk in range(5):
            pltpu.sync_copy(accdeg.at[pl.ds(48000 + kk * 400, 400)],
                            vbuf.at[pl.ds(0, 400)])
            pltpu.sync_copy(vbuf.at[pl.ds(0, 400)],
                            deg.at[pl.ds(c * _HALF + 48000 + kk * 400, 400)])


@functools.partial(
    pl.kernel,
    out_type=jax.ShapeDtypeStruct((_NPAD, _HHID), jnp.float32),
    mesh=_mesh,
    compiler_params=pltpu.CompilerParams(use_tc_tiling_on_sc=False, needs_layout_passes=False),
    scratch_types=[
        pltpu.VMEM((8, 128), jnp.int32),
        pltpu.VMEM((8, 128), jnp.int32),
        pltpu.VMEM((8, 128), jnp.int32),
        pltpu.VMEM((8, 128), jnp.int32),
        pltpu.VMEM((1024, _HHID), jnp.float32),
        pltpu.VMEM((1024, _HHID), jnp.float32),
        pltpu.VMEM((400, _HHID), jnp.float32),
        pltpu.VMEM_SHARED((_ACC_ROWS, _HHID), jnp.float32),
        pltpu.SemaphoreType.DMA,
        pltpu.SemaphoreType.DMA,
    ],
)
def _sc_scatter(p, src2d, lidx0, lidx1, zerosf, acc_out,
                srcA, srcB, lidxA, lidxB, rowsA, rowsB, vbuf, accf,
                gsem, ssem):
    c = lax.axis_index("c")
    s = lax.axis_index("s")
    pltpu.sync_copy(zerosf, vbuf)
    for kk in range(8):
        pltpu.sync_copy(vbuf, accf.at[pl.ds(s * 3200 + kk * 400, 400)])
    plsc.subcore_barrier()

    def load_idx(row0, srcb, lidxb):
        pltpu.sync_copy(src2d.at[pl.ds(row0, 8)], srcb)

        @pl.when(c == 0)
        def _():
            pltpu.sync_copy(lidx0.at[pl.ds(row0, 8)], lidxb)

        @pl.when(c == 1)
        def _():
            pltpu.sync_copy(lidx1.at[pl.ds(row0, 8)], lidxb)

    def issue_g(srcb, rows):
        return [
            pltpu.async_copy(p.at[srcb.at[j]], rows.at[pl.ds(j * 128, 128)], gsem)
            for j in range(8)
        ]

    def wait_g(srcb, rows):
        for j in range(8):
            pltpu.make_async_copy(p.at[srcb.at[j]],
                                  rows.at[pl.ds(j * 128, 128)], gsem).wait()

    def issue_s(rows, lidxb):
        return [
            pltpu.async_copy(rows.at[pl.ds(j * 128, 128)],
                             accf.at[lidxb.at[j]], ssem, add=True)
            for j in range(8)
        ]

    load_idx(s * _RPW, srcA, lidxA)
    issue_g(srcA, rowsA)

    def pair(t, carry):
        rowA = s * _RPW + (2 * t) * 8
        wait_g(srcA, rowsA)
        scA = issue_s(rowsA, lidxA)
        load_idx(rowA + 8, srcB, lidxB)
        issue_g(srcB, rowsB)
        for cp in scA:
            cp.wait()
        wait_g(srcB, rowsB)
        scB = issue_s(rowsB, lidxB)

        @pl.when(t < (_NCHUNK // 2 - 1))
        def _():
            load_idx(rowA + 16, srcA, lidxA)
            issue_g(srcA, rowsA)

        for cp in scB:
            cp.wait()
        return carry

    lax.fori_loop(0, _NCHUNK // 2, pair, 0)
    plsc.subcore_barrier()

    @pl.when(s < 15)
    def _():
        for kk in range(8):
            pltpu.sync_copy(accf.at[pl.ds(s * 3200 + kk * 400, 400)], vbuf)
            pltpu.sync_copy(vbuf, acc_out.at[pl.ds(c * _HALF + s * 3200 + kk * 400, 400)])

    @pl.when(s == 15)
    def _():
        for kk in range(5):
            pltpu.sync_copy(accf.at[pl.ds(48000 + kk * 400, 400)], vbuf)
            pltpu.sync_copy(vbuf, acc_out.at[pl.ds(c * _HALF + 48000 + kk * 400, 400)])


@functools.partial(
    pl.kernel,
    out_type=jax.ShapeDtypeStruct((32 * _PROWS * _PCOLS,), jnp.float32),
    mesh=_mesh,
    compiler_params=pltpu.CompilerParams(use_tc_tiling_on_sc=False, needs_layout_passes=False),
    scratch_types=[
        pltpu.VMEM((128, _HID), jnp.float32),
        pltpu.VMEM((128,), jnp.int32),
        pltpu.VMEM((_PROWS * _PCOLS,), jnp.float32),
    ],
)
def _sc_pool(h3, batch1d, zerosp, part, hbuf, bbuf, accp):
    c = lax.axis_index("c")
    s = lax.axis_index("s")
    w = s * 2 + c
    pltpu.sync_copy(zerosp, accp)
    lanes = lax.iota(jnp.int32, 16)
    cntv = jnp.where(lanes == 0, 1.0, 0.0)

    def chunk(i, carry):
        pltpu.sync_copy(h3.at[pl.ds(w * 3200 + i * 128, 128)], hbuf)
        pltpu.sync_copy(batch1d.at[pl.ds(w * 3200 + i * 128, 128)], bbuf)
        for k in range(8):
            bvec = bbuf[pl.ds(k * 16, 16)]
            for j in range(16):
                n = k * 16 + j
                b_s = jnp.sum(jnp.where(lanes == j, bvec, 0))
                idx0 = b_s * _PCOLS + lanes
                plsc.addupdate_scatter(accp, [idx0], hbuf[n, pl.ds(0, 16)])
                plsc.addupdate_scatter(accp, [idx0 + 16], hbuf[n, pl.ds(16, 16)])
                plsc.addupdate_scatter(accp, [idx0 + 32], cntv)
        return carry

    lax.fori_loop(0, _PPW, chunk, 0)
    pltpu.sync_copy(accp, part.at[pl.ds(w * (_PROWS * _PCOLS), _PROWS * _PCOLS)])


_BLK = 3200


def _tc0_body(deg_ref, h_ref, w_ref, pl_ref, ph_ref, dinv_ref):
    i = pl.program_id(0)
    gid = i * _BLK + jax.lax.broadcasted_iota(jnp.int32, (_BLK, 1), 0)
    real = gid < _N
    dv = jnp.where(real, lax.rsqrt(deg_ref[:, :1] + 1.0), 0.0)
    p = jnp.dot(h_ref[...], w_ref[...],
                preferred_element_type=jnp.float32) * dv
    pl_ref[...] = p[:, :_HHID]
    ph_ref[...] = p[:, _HHID:]
    dinv_ref[...] = dv


def _tc0(h0, deg, W1):
    return pl.pallas_call(
        _tc0_body,
        grid=(_NPAD // _BLK,),
        in_specs=[
            pl.BlockSpec((_BLK, _HHID), lambda i: (i, 0)),
            pl.BlockSpec((_BLK, _HID), lambda i: (i, 0)),
            pl.BlockSpec((_HID, _HID), lambda i: (0, 0)),
        ],
        out_specs=[
            pl.BlockSpec((_BLK, _HHID), lambda i: (i, 0)),
            pl.BlockSpec((_BLK, _HHID), lambda i: (i, 0)),
            pl.BlockSpec((_BLK, 1), lambda i: (i, 0)),
        ],
        out_shape=[
            jax.ShapeDtypeStruct((_NPAD, _HHID), jnp.float32),
            jax.ShapeDtypeStruct((_NPAD, _HHID), jnp.float32),
            jax.ShapeDtypeStruct((_NPAD, 1), jnp.float32),
        ],
    )(deg, h0, W1)


def _tc_mid_body(al_ref, ah_ref, pl_ref, ph_ref, dinv_ref, b_ref, w_ref,
                 ol_ref, oh_ref):
    dv = dinv_ref[...]
    acc = jnp.concatenate([al_ref[...], ah_ref[...]], axis=1)
    p = jnp.concatenate([pl_ref[...], ph_ref[...]], axis=1)
    h = jnp.maximum(dv * (acc + p) + b_ref[...], 0.0)
    pn = jnp.dot(h, w_ref[...], preferred_element_type=jnp.float32) * dv
    ol_ref[...] = pn[:, :_HHID]
    oh_ref[...] = pn[:, _HHID:]


def _tc_mid(al, ah, p_lo, p_hi, dinv, b, W):
    return pl.pallas_call(
        _tc_mid_body,
        grid=(_NPAD // _BLK,),
        in_specs=[
            pl.BlockSpec((_BLK, _HHID), lambda i: (i, 0)),
            pl.BlockSpec((_BLK, _HHID), lambda i: (i, 0)),
            pl.BlockSpec((_BLK, _HHID), lambda i: (i, 0)),
            pl.BlockSpec((_BLK, _HHID), lambda i: (i, 0)),
            pl.BlockSpec((_BLK, 1), lambda i: (i, 0)),
            pl.BlockSpec((1, _HID), lambda i: (0, 0)),
            pl.BlockSpec((_HID, _HID), lambda i: (0, 0)),
        ],
        out_specs=[
            pl.BlockSpec((_BLK, _HHID), lambda i: (i, 0)),
            pl.BlockSpec((_BLK, _HHID), lambda i: (i, 0)),
        ],
        out_shape=[
            jax.ShapeDtypeStruct((_NPAD, _HHID), jnp.float32),
            jax.ShapeDtypeStruct((_NPAD, _HHID), jnp.float32),
        ],
    )(al, ah, p_lo, p_hi, dinv, b, W)


def _tc_h3_body(al_ref, ah_ref, pl_ref, ph_ref, dinv_ref, b_ref, out_ref):
    acc = jnp.concatenate([al_ref[...], ah_ref[...]], axis=1)
    p = jnp.concatenate([pl_ref[...], ph_ref[...]], axis=1)
    out_ref[...] = dinv_ref[...] * (acc + p) + b_ref[...]


def _tc_h3(al, ah, p_lo, p_hi, dinv, b):
    return pl.pallas_call(
        _tc_h3_body,
        grid=(_NPAD // _BLK,),
        in_specs=[
            pl.BlockSpec((_BLK, _HHID), lambda i: (i, 0)),
            pl.BlockSpec((_BLK, _HHID), lambda i: (i, 0)),
            pl.BlockSpec((_BLK, _HHID), lambda i: (i, 0)),
            pl.BlockSpec((_BLK, _HHID), lambda i: (i, 0)),
            pl.BlockSpec((_BLK, 1), lambda i: (i, 0)),
            pl.BlockSpec((1, _HID), lambda i: (0, 0)),
        ],
        out_specs=pl.BlockSpec((_BLK, _HID), lambda i: (i, 0)),
        out_shape=jax.ShapeDtypeStruct((_NPAD, _HID), jnp.float32),
    )(al, ah, p_lo, p_hi, dinv, b)


def _tc4_body(part_ref, lw_ref, lb_ref, out_ref):
    t = jnp.sum(part_ref[...], axis=0)
    sums = t[:64, :_HID]
    cnt = t[:64, _HID:_HID + 1]
    pooled = sums / jnp.maximum(cnt, 1.0)
    out_ref[...] = jnp.dot(pooled, lw_ref[...],
                           preferred_element_type=jnp.float32) + lb_ref[...]


def _tc4(part, lin_W, lin_b):
    return pl.pallas_call(
        _tc4_body,
        out_shape=jax.ShapeDtypeStruct((64, _OUT), jnp.float32),
    )(part, lin_W, lin_b)


def kernel(x, edge_index, edge_attr, batch, embed,
           W1, b1, W2, b2, W3, b3, lin_W, lin_b):
    del edge_attr
    xp = jnp.concatenate([x[:, 0], jnp.zeros((_NPAD - _N,), jnp.int32)])
    src = edge_index[0]
    dst = edge_index[1]
    pad_e = _EPAD - src.shape[0]
    src2d = jnp.concatenate(
        [src, jnp.zeros((pad_e,), jnp.int32)]).reshape(_E2D, 128)
    dst2d = jnp.concatenate(
        [dst, jnp.full((pad_e,), 1 << 20, jnp.int32)]).reshape(_E2D, 128)
    batchp = jnp.concatenate([batch, jnp.full((_NPAD - _N,), 64, jnp.int32)])
    ones1 = jnp.ones((128, _HHID), jnp.float32)
    zeros1 = jnp.zeros((640, _HHID), jnp.float32)
    zerosf = jnp.zeros((400, _HHID), jnp.float32)
    zerosp = jnp.zeros((_PROWS * _PCOLS,), jnp.float32)

    h0, deg, lidx0, lidx1 = _sc_embed_deg(embed, xp, dst2d, ones1, zeros1)
    p1l, p1h, dinv = _tc0(h0, deg, W1)
    a1l = _sc_scatter(p1l, src2d, lidx0, lidx1, zerosf)
    a1h = _sc_scatter(p1h, src2d, lidx0, lidx1, zerosf)
    p2l, p2h = _tc_mid(a1l, a1h, p1l, p1h, dinv, b1.reshape(1, _HID), W2)
    a2l = _sc_scatter(p2l, src2d, lidx0, lidx1, zerosf)
    a2h = _sc_scatter(p2h, src2d, lidx0, lidx1, zerosf)
    p3l, p3h = _tc_mid(a2l, a2h, p2l, p2h, dinv, b2.reshape(1, _HID), W3)
    a3l = _sc_scatter(p3l, src2d, lidx0, lidx1, zerosf)
    a3h = _sc_scatter(p3h, src2d, lidx0, lidx1, zerosf)
    h3 = _tc_h3(a3l, a3h, p3l, p3h, dinv, b3.reshape(1, _HID))
    part = _sc_pool(h3, batchp, zerosp)
    out = _tc4(part.reshape(32, _PROWS, _PCOLS), lin_W, lin_b.reshape(1, _OUT))
    return out


# speedup vs baseline: 15.5981x; 2.3244x over previous
import functools

import jax
import jax.numpy as jnp
from jax import lax
from jax.experimental import pallas as pl
from jax.experimental.pallas import tpu as pltpu
from jax.experimental.pallas import tpu_sc as plsc

_N = 100000
_NPAD = 102400
_HID = 32
_OUT = 16
_E2D = 12544
_EPAD = _E2D * 128
_HALF = 50000
_DUMP = 50000
_ACC_ROWS = 51200
_RPW = _E2D // 16
_NCHUNK = _RPW // 8
_PPW = 25
_PROWS = 72
_PCOLS = 48
_HHID = 16

_mesh = plsc.VectorSubcoreMesh(core_axis_name="c", subcore_axis_name="s")


@functools.partial(
    pl.kernel,
    out_type=[
        jax.ShapeDtypeStruct((_NPAD, _HID), jnp.float32),
        jax.ShapeDtypeStruct((_NPAD, _HHID), jnp.float32),
        jax.ShapeDtypeStruct((_E2D, 128), jnp.int32),
        jax.ShapeDtypeStruct((_E2D, 128), jnp.int32),
    ],
    mesh=_mesh,
    compiler_params=pltpu.CompilerParams(use_tc_tiling_on_sc=False, needs_layout_passes=False),
    scratch_types=[
        pltpu.VMEM((_PPW * 128,), jnp.int32),
        pltpu.VMEM((640, _HID), jnp.float32),
        pltpu.VMEM((8, 128), jnp.int32),
        pltpu.VMEM((8, 128), jnp.int32),
        pltpu.VMEM((128, _HHID), jnp.float32),
        pltpu.VMEM((640, _HHID), jnp.float32),
        pltpu.VMEM_SHARED((_ACC_ROWS, _HHID), jnp.float32),
        pltpu.SemaphoreType.DMA,
    ],
)
def _sc_embed_deg(embed, x1d, dst2d, ones1, zeros1,
                  h0, deg, lidx0, lidx1,
                  idxv, rows, dstb, lidxb, onesv, vbuf, accdeg, sem):
    c = lax.axis_index("c")
    s = lax.axis_index("s")
    w = s * 2 + c

    pltpu.sync_copy(x1d.at[pl.ds(w * 3200, 3200)], idxv)

    def eround(r, carry):
        cps = [
            pltpu.async_copy(embed.at[idxv.at[pl.ds(r * 640 + j * 128, 128)]],
                             rows.at[pl.ds(j * 128, 128)], sem)
            for j in range(5)
        ]
        for cp in cps:
            cp.wait()
        pltpu.sync_copy(rows, h0.at[pl.ds(w * 3200 + r * 640, 640)])
        return carry

    lax.fori_loop(0, 5, eround, 0)

    pltpu.sync_copy(ones1, onesv)
    pltpu.sync_copy(zeros1, vbuf)
    for kk in range(5):
        pltpu.sync_copy(vbuf, accdeg.at[pl.ds(s * 3200 + kk * 640, 640)])
    plsc.subcore_barrier()
    base = c * _HALF
    dumpv = _DUMP + s * 16 + lax.iota(jnp.int32, 16)

    def chunk(i, carry):
        row0 = s * _RPW + i * 8
        pltpu.sync_copy(dst2d.at[pl.ds(row0, 8)], dstb)
        for j in range(8):
            for k in range(8):
                d = dstb[j, pl.ds(k * 16, 16)]
                m = (d >= base) & (d < base + _HALF)
                lidxb[j, pl.ds(k * 16, 16)] = jnp.where(m, d - base, dumpv)

        @pl.when(c == 0)
        def _():
            pltpu.sync_copy(lidxb, lidx0.at[pl.ds(row0, 8)])

        @pl.when(c == 1)
        def _():
            pltpu.sync_copy(lidxb, lidx1.at[pl.ds(row0, 8)])

        for j in range(8):
            pltpu.sync_copy(onesv, accdeg.at[lidxb.at[j]], add=True)
        return carry

    lax.fori_loop(0, _NCHUNK, chunk, 0)
    plsc.subcore_barrier()

    @pl.when(s < 15)
    def _():
        for kk in range(5):
            pltpu.sync_copy(accdeg.at[pl.ds(s * 3200 + kk * 640, 640)], vbuf)
            pltpu.sync_copy(vbuf, deg.at[pl.ds(c * _HALF + s * 3200 + kk * 640, 640)])

    @pl.when(s == 15)
    def _():
        for kk in range(5):
            pltpu.sync_copy(accdeg.at[pl.ds(48000 + kk * 400, 400)],
                            vbuf.at[pl.ds(0, 400)])
            pltpu.sync_copy(vbuf.at[pl.ds(0, 400)],
                            deg.at[pl.ds(c * _HALF + 48000 + kk * 400, 400)])


@functools.partial(
    pl.kernel,
    out_type=jax.ShapeDtypeStruct((_NPAD, _HHID), jnp.float32),
    mesh=_mesh,
    compiler_params=pltpu.CompilerParams(use_tc_tiling_on_sc=False, needs_layout_passes=False),
    scratch_types=[
        pltpu.VMEM((8, 128), jnp.int32),
        pltpu.VMEM((8, 128), jnp.int32),
        pltpu.VMEM((8, 128), jnp.int32),
        pltpu.VMEM((8, 128), jnp.int32),
        pltpu.VMEM((1024, _HHID), jnp.float32),
        pltpu.VMEM((1024, _HHID), jnp.float32),
        pltpu.VMEM((400, _HHID), jnp.float32),
        pltpu.VMEM_SHARED((_ACC_ROWS, _HHID), jnp.float32),
        pltpu.SemaphoreType.DMA,
        pltpu.SemaphoreType.DMA,
    ],
)
def _sc_scatter(p, src2d, lidx0, lidx1, zerosf, acc_out,
                srcA, srcB, lidxA, lidxB, rowsA, rowsB, vbuf, accf,
                gsem, ssem):
    c = lax.axis_index("c")
    s = lax.axis_index("s")
    pltpu.sync_copy(zerosf, vbuf)
    for kk in range(8):
        pltpu.sync_copy(vbuf, accf.at[pl.ds(s * 3200 + kk * 400, 400)])
    plsc.subcore_barrier()

    def load_idx(row0, srcb, lidxb):
        pltpu.sync_copy(src2d.at[pl.ds(row0, 8)], srcb)

        @pl.when(c == 0)
        def _():
            pltpu.sync_copy(lidx0.at[pl.ds(row0, 8)], lidxb)

        @pl.when(c == 1)
        def _():
            pltpu.sync_copy(lidx1.at[pl.ds(row0, 8)], lidxb)

    def issue_g(srcb, rows):
        return [
            pltpu.async_copy(p.at[srcb.at[j]], rows.at[pl.ds(j * 128, 128)], gsem)
            for j in range(8)
        ]

    def wait_g(srcb, rows):
        for j in range(8):
            pltpu.make_async_copy(p.at[srcb.at[j]],
                                  rows.at[pl.ds(j * 128, 128)], gsem).wait()

    def issue_s(rows, lidxb):
        return [
            pltpu.async_copy(rows.at[pl.ds(j * 128, 128)],
                             accf.at[lidxb.at[j]], ssem, add=True)
            for j in range(8)
        ]

    load_idx(s * _RPW, srcA, lidxA)
    issue_g(srcA, rowsA)

    def pair(t, carry):
        rowA = s * _RPW + (2 * t) * 8
        wait_g(srcA, rowsA)
        scA = issue_s(rowsA, lidxA)
        load_idx(rowA + 8, srcB, lidxB)
        issue_g(srcB, rowsB)
        for cp in scA:
            cp.wait()
        wait_g(srcB, rowsB)
        scB = issue_s(rowsB, lidxB)

        @pl.when(t < (_NCHUNK // 2 - 1))
        def _():
            load_idx(rowA + 16, srcA, lidxA)
            issue_g(srcA, rowsA)

        for cp in scB:
            cp.wait()
        return carry

    lax.fori_loop(0, _NCHUNK // 2, pair, 0)
    plsc.subcore_barrier()

    @pl.when(s < 15)
    def _():
        for kk in range(8):
            pltpu.sync_copy(accf.at[pl.ds(s * 3200 + kk * 400, 400)], vbuf)
            pltpu.sync_copy(vbuf, acc_out.at[pl.ds(c * _HALF + s * 3200 + kk * 400, 400)])

    @pl.when(s == 15)
    def _():
        for kk in range(5):
            pltpu.sync_copy(accf.at[pl.ds(48000 + kk * 400, 400)], vbuf)
            pltpu.sync_copy(vbuf, acc_out.at[pl.ds(c * _HALF + 48000 + kk * 400, 400)])


@functools.partial(
    pl.kernel,
    out_type=jax.ShapeDtypeStruct((32 * _PROWS * _PCOLS,), jnp.float32),
    mesh=_mesh,
    compiler_params=pltpu.CompilerParams(use_tc_tiling_on_sc=False, needs_layout_passes=False),
    scratch_types=[
        pltpu.VMEM((128, _HID), jnp.float32),
        pltpu.VMEM((128,), jnp.int32),
        pltpu.VMEM((_PROWS * _PCOLS,), jnp.float32),
    ],
)
def _sc_pool(h3, batch1d, zerosp, part, hbuf, bbuf, accp):
    c = lax.axis_index("c")
    s = lax.axis_index("s")
    w = s * 2 + c
    pltpu.sync_copy(zerosp, accp)
    lanes = lax.iota(jnp.int32, 16)
    cntv = jnp.where(lanes == 0, 1.0, 0.0)

    def chunk(i, carry):
        pltpu.sync_copy(h3.at[pl.ds(w * 3200 + i * 128, 128)], hbuf)
        pltpu.sync_copy(batch1d.at[pl.ds(w * 3200 + i * 128, 128)], bbuf)
        for k in range(8):
            bvec = bbuf[pl.ds(k * 16, 16)]
            for j in range(16):
                n = k * 16 + j
                b_s = jnp.sum(jnp.where(lanes == j, bvec, 0))
                idx0 = b_s * _PCOLS + lanes
                plsc.addupdate_scatter(accp, [idx0], hbuf[n, pl.ds(0, 16)])
                plsc.addupdate_scatter(accp, [idx0 + 16], hbuf[n, pl.ds(16, 16)])
                plsc.addupdate_scatter(accp, [idx0 + 32], cntv)
        return carry

    lax.fori_loop(0, _PPW, chunk, 0)
    pltpu.sync_copy(accp, part.at[pl.ds(w * (_PROWS * _PCOLS), _PROWS * _PCOLS)])


_BLK = 3200


def _tc0_body(deg_ref, h_ref, w_ref, pl_ref, ph_ref, dinv_ref):
    i = pl.program_id(0)
    gid = i * _BLK + jax.lax.broadcasted_iota(jnp.int32, (_BLK, 1), 0)
    real = gid < _N
    dv = jnp.where(real, lax.rsqrt(deg_ref[:, :1] + 1.0), 0.0)
    p = jnp.dot(h_ref[...], w_ref[...],
                preferred_element_type=jnp.float32) * dv
    pl_ref[...] = p[:, :_HHID]
    ph_ref[...] = p[:, _HHID:]
    dinv_ref[...] = dv


def _tc0(h0, deg, W1):
    return pl.pallas_call(
        _tc0_body,
        grid=(_NPAD // _BLK,),
        in_specs=[
            pl.BlockSpec((_BLK, _HHID), lambda i: (i, 0)),
            pl.BlockSpec((_BLK, _HID), lambda i: (i, 0)),
            pl.BlockSpec((_HID, _HID), lambda i: (0, 0)),
        ],
        out_specs=[
            pl.BlockSpec((_BLK, _HHID), lambda i: (i, 0)),
            pl.BlockSpec((_BLK, _HHID), lambda i: (i, 0)),
            pl.BlockSpec((_BLK, 1), lambda i: (i, 0)),
        ],
        out_shape=[
            jax.ShapeDtypeStruct((_NPAD, _HHID), jnp.float32),
            jax.ShapeDtypeStruct((_NPAD, _HHID), jnp.float32),
            jax.ShapeDtypeStruct((_NPAD, 1), jnp.float32),
        ],
    )(deg, h0, W1)


def _tc_mid_body(al_ref, ah_ref, pl_ref, ph_ref, dinv_ref, b_ref, w_ref,
                 ol_ref, oh_ref):
    dv = dinv_ref[...]
    acc = jnp.concatenate([al_ref[...], ah_ref[...]], axis=1)
    p = jnp.concatenate([pl_ref[...], ph_ref[...]], axis=1)
    h = jnp.maximum(dv * (acc + p) + b_ref[...], 0.0)
    pn = jnp.dot(h, w_ref[...], preferred_element_type=jnp.float32) * dv
    ol_ref[...] = pn[:, :_HHID]
    oh_ref[...] = pn[:, _HHID:]


def _tc_mid(al, ah, p_lo, p_hi, dinv, b, W):
    return pl.pallas_call(
        _tc_mid_body,
        grid=(_NPAD // _BLK,),
        in_specs=[
            pl.BlockSpec((_BLK, _HHID), lambda i: (i, 0)),
            pl.BlockSpec((_BLK, _HHID), lambda i: (i, 0)),
            pl.BlockSpec((_BLK, _HHID), lambda i: (i, 0)),
            pl.BlockSpec((_BLK, _HHID), lambda i: (i, 0)),
            pl.BlockSpec((_BLK, 1), lambda i: (i, 0)),
            pl.BlockSpec((1, _HID), lambda i: (0, 0)),
            pl.BlockSpec((_HID, _HID), lambda i: (0, 0)),
        ],
        out_specs=[
            pl.BlockSpec((_BLK, _HHID), lambda i: (i, 0)),
            pl.BlockSpec((_BLK, _HHID), lambda i: (i, 0)),
        ],
        out_shape=[
            jax.ShapeDtypeStruct((_NPAD, _HHID), jnp.float32),
            jax.ShapeDtypeStruct((_NPAD, _HHID), jnp.float32),
        ],
    )(al, ah, p_lo, p_hi, dinv, b, W)


def _tc_h3_body(al_ref, ah_ref, pl_ref, ph_ref, dinv_ref, b_ref, out_ref):
    acc = jnp.concatenate([al_ref[...], ah_ref[...]], axis=1)
    p = jnp.concatenate([pl_ref[...], ph_ref[...]], axis=1)
    out_ref[...] = dinv_ref[...] * (acc + p) + b_ref[...]


def _tc_h3(al, ah, p_lo, p_hi, dinv, b):
    return pl.pallas_call(
        _tc_h3_body,
        grid=(_NPAD // _BLK,),
        in_specs=[
            pl.BlockSpec((_BLK, _HHID), lambda i: (i, 0)),
            pl.BlockSpec((_BLK, _HHID), lambda i: (i, 0)),
            pl.BlockSpec((_BLK, _HHID), lambda i: (i, 0)),
            pl.BlockSpec((_BLK, _HHID), lambda i: (i, 0)),
            pl.BlockSpec((_BLK, 1), lambda i: (i, 0)),
            pl.BlockSpec((1, _HID), lambda i: (0, 0)),
        ],
        out_specs=pl.BlockSpec((_BLK, _HID), lambda i: (i, 0)),
        out_shape=jax.ShapeDtypeStruct((_NPAD, _HID), jnp.float32),
    )(al, ah, p_lo, p_hi, dinv, b)


def _tc4_body(part_ref, lw_ref, lb_ref, out_ref):
    t = jnp.sum(part_ref[...], axis=0)
    sums = t[:64, :_HID]
    cnt = t[:64, _HID:_HID + 1]
    pooled = sums / jnp.maximum(cnt, 1.0)
    out_ref[...] = jnp.dot(pooled, lw_ref[...],
                           preferred_element_type=jnp.float32) + lb_ref[...]


def _tc4(part, lin_W, lin_b):
    return pl.pallas_call(
        _tc4_body,
        out_shape=jax.ShapeDtypeStruct((64, _OUT), jnp.float32),
    )(part, lin_W, lin_b)


def kernel(x, edge_index, edge_attr, batch, embed,
           W1, b1, W2, b2, W3, b3, lin_W, lin_b):
    del edge_attr
    xp = jnp.concatenate([x[:, 0], jnp.zeros((_NPAD - _N,), jnp.int32)])
    src = edge_index[0]
    dst = edge_index[1]
    pad_e = _EPAD - src.shape[0]
    src2d = jnp.concatenate(
        [src, jnp.zeros((pad_e,), jnp.int32)]).reshape(_E2D, 128)
    dst2d = jnp.concatenate(
        [dst, jnp.full((pad_e,), 1 << 20, jnp.int32)]).reshape(_E2D, 128)
    batchp = jnp.concatenate([batch, jnp.full((_NPAD - _N,), 64, jnp.int32)])
    ones1 = jnp.ones((128, _HHID), jnp.float32)
    zeros1 = jnp.zeros((640, _HHID), jnp.float32)
    zerosf = jnp.zeros((400, _HHID), jnp.float32)
    zerosp = jnp.zeros((_PROWS * _PCOLS,), jnp.float32)

    h0, deg, lidx0, lidx1 = _sc_embed_deg(embed, xp, dst2d, ones1, zeros1)
    p1l, p1h, dinv = _tc0(h0, deg, W1)
    a1l = _sc_scatter(p1l, src2d, lidx0, lidx1, zerosf)
    a1h = _sc_scatter(p1h, src2d, lidx0, lidx1, zerosf)
    p2l, p2h = _tc_mid(a1l, a1h, p1l, p1h, dinv, b1.reshape(1, _HID), W2)
    a2l = _sc_scatter(p2l, src2d, lidx0, lidx1, zerosf)
    a2h = _sc_scatter(p2h, src2d, lidx0, lidx1, zerosf)
    p3l, p3h = _tc_mid(a2l, a2h, p2l, p2h, dinv, b2.reshape(1, _HID), W3)
    a3l = _sc_scatter(p3l, src2d, lidx0, lidx1, zerosf)
    a3h = _sc_scatter(p3h, src2d, lidx0, lidx1, zerosf)
    h3 = _tc_h3(a3l, a3h, p3l, p3h, dinv, b3.reshape(1, _HID))
    part = _sc_pool(h3, batchp, zerosp)
    out = _tc4(part.reshape(32, _PROWS, _PCOLS), lin_W, lin_b.reshape(1, _OUT))
    return out
